# baseline (speedup 1.0000x reference)
import jax
import jax.numpy as jnp
from jax import lax
from jax.experimental import pallas as pl
from jax.experimental.pallas import tpu as pltpu

HID = 512
OUT = 64

_VMEM_LIMIT = 34 * 1024 * 1024
_ANYVMEM = pl.BlockSpec(memory_space=pltpu.MemorySpace.VMEM)


def _recurrence(xp_ref, whh, is_bwd, nT, H, hseq_ref):
    idx0 = jnp.where(is_bwd, nT - 1, 0)
    g0 = xp_ref[idx0]
    c = jax.nn.sigmoid(g0[:, 0 * H:1 * H]) * jnp.tanh(g0[:, 2 * H:3 * H])
    h = jax.nn.sigmoid(g0[:, 3 * H:4 * H]) * jnp.tanh(c)
    hseq_ref[idx0] = h

    for t in range(1, nT):
        idx = jnp.where(is_bwd, nT - 1 - t, t)
        gates = xp_ref[idx] + jnp.dot(h.astype(jnp.bfloat16), whh,
                                      preferred_element_type=jnp.float32)
        i_g = jax.nn.sigmoid(gates[:, 0 * H:1 * H])
        f_g = jax.nn.sigmoid(gates[:, 1 * H:2 * H])
        g_g = jnp.tanh(gates[:, 2 * H:3 * H])
        o_g = jax.nn.sigmoid(gates[:, 3 * H:4 * H])
        c = f_g * c + i_g * g_g
        h = o_g * jnp.tanh(c)
        hseq_ref[idx] = h
    return h, c


def _l0_kernel(tokens_ref, emb_ref, wihT_ref, whhT_ref, b_ref,
               hseq_ref, hlast_ref, clast_ref, xp_ref):
    nB, nT = tokens_ref.shape
    V, E = emb_ref.shape
    H = hlast_ref.shape[-1]
    is_bwd = pl.program_id(0) == 1

    tok_f = tokens_ref[...].astype(jnp.float32)
    eye_b = (lax.broadcasted_iota(jnp.int32, (nB, nB), 0)
             == lax.broadcasted_iota(jnp.int32, (nB, nB), 1)).astype(jnp.float32)
    tok_t = lax.dot_general(tok_f, eye_b, (((0,), (0,)), ((), ())),
                            preferred_element_type=jnp.float32)
    vids = lax.broadcasted_iota(jnp.int32, (nT, nB, V), 2)
    oh = (vids == tok_t.astype(jnp.int32)[:, :, None]).astype(jnp.bfloat16)

    x2d = jnp.dot(oh.reshape(nT * nB, V), emb_ref[...].astype(jnp.bfloat16),
                  preferred_element_type=jnp.float32).astype(jnp.bfloat16)
    xp_ref[...] = (jnp.dot(x2d, wihT_ref[...],
                           preferred_element_type=jnp.float32)
                   + b_ref[...]).reshape(nT, nB, 4 * H)

    h, c = _recurrence(xp_ref, whhT_ref[...], is_bwd, nT, H, hseq_ref)
    hlast_ref[...] = h
    clast_ref[...] = c


def _l1_kernel(x_ref, wihT_ref, whhT_ref, b_ref, h0_ref, c0_ref,
               hseq_ref, hn_ref, cn_ref, xp_ref):
    nT, nB, in_dim = x_ref.shape
    H = h0_ref.shape[-1]
    is_bwd = pl.program_id(0) == 1

    x2d = x_ref[...].reshape(nT * nB, in_dim).astype(jnp.bfloat16)
    xp_ref[...] = (jnp.dot(x2d, wihT_ref[...],
                           preferred_element_type=jnp.float32)
                   + b_ref[...]).reshape(nT, nB, 4 * H)

    h, c = _recurrence(xp_ref, whhT_ref[...], is_bwd, nT, H, hseq_ref)
    hn_ref[0] = h0_ref[...]
    hn_ref[1] = h
    cn_ref[0] = c0_ref[...]
    cn_ref[1] = c


def _l0_call(tokens, embedding, wihT, whhT, b, nT, nB):
    return pl.pallas_call(
        _l0_kernel,
        out_shape=(jax.ShapeDtypeStruct((nT, nB, 2 * HID), jnp.float32),
                   jax.ShapeDtypeStruct((2, nB, HID), jnp.float32),
                   jax.ShapeDtypeStruct((2, nB, HID), jnp.float32)),
        grid=(2,),
        in_specs=[
            pl.BlockSpec((nB, nT), lambda d: (0, 0)),
            pl.BlockSpec(embedding.shape, lambda d: (0, 0)),
            pl.BlockSpec((None,) + wihT.shape[1:], lambda d: (d, 0, 0)),
            pl.BlockSpec((None, HID, 4 * HID), lambda d: (d, 0, 0)),
            pl.BlockSpec((None, 1, 4 * HID), lambda d: (d, 0, 0)),
        ],
        out_specs=(
            pl.BlockSpec((nT, nB, HID), lambda d: (0, 0, d)),
            pl.BlockSpec((None, nB, HID), lambda d: (d, 0, 0)),
            pl.BlockSpec((None, nB, HID), lambda d: (d, 0, 0)),
        ),
        scratch_shapes=[pltpu.VMEM((nT, nB, 4 * HID), jnp.float32)],
        compiler_params=pltpu.CompilerParams(
            dimension_semantics=("parallel",),
            vmem_limit_bytes=_VMEM_LIMIT),
    )(tokens, embedding, wihT, whhT, b)


def _l1_call(x_tbf, wihT, whhT, b, h0, c0):
    nT, nB, in_dim = x_tbf.shape
    return pl.pallas_call(
        _l1_kernel,
        out_shape=(jax.ShapeDtypeStruct((nT, nB, 2 * HID), jnp.float32),
                   jax.ShapeDtypeStruct((2, 2, nB, HID), jnp.float32),
                   jax.ShapeDtypeStruct((2, 2, nB, HID), jnp.float32)),
        grid=(2,),
        in_specs=[
            pl.BlockSpec((nT, nB, in_dim), lambda d: (0, 0, 0)),
            pl.BlockSpec((None, in_dim, 4 * HID), lambda d: (d, 0, 0)),
            pl.BlockSpec((None, HID, 4 * HID), lambda d: (d, 0, 0)),
            pl.BlockSpec((None, 1, 4 * HID), lambda d: (d, 0, 0)),
            pl.BlockSpec((None, nB, HID), lambda d: (d, 0, 0)),
            pl.BlockSpec((None, nB, HID), lambda d: (d, 0, 0)),
        ],
        out_specs=(
            pl.BlockSpec((nT, nB, HID), lambda d: (0, 0, d)),
            pl.BlockSpec((2, None, nB, HID), lambda d: (0, d, 0, 0)),
            pl.BlockSpec((2, None, nB, HID), lambda d: (0, d, 0, 0)),
        ),
        scratch_shapes=[pltpu.VMEM((nT, nB, 4 * HID), jnp.float32)],
        compiler_params=pltpu.CompilerParams(
            dimension_semantics=("parallel",),
            vmem_limit_bytes=_VMEM_LIMIT),
    )(x_tbf, wihT, whhT, b, h0, c0)


def _head_kernel(out_ref, hlast_ref,
                 waT_ref, ba_ref, wc1T_ref, wc2T_ref, bc_ref, wlT_ref, bl_ref,
                 whpT_ref, bhp_ref, wopT_ref, bop_ref,
                 wa1aT_ref, wa1bT_ref, ba1_ref, wa2T_ref, ba2_ref,
                 logits_ref, score_ref, ent_ref, sim_ref, disp_ref):
    eps = 1e-9
    nT, nB, D2 = out_ref.shape
    nO = logits_ref.shape[-1]
    out = out_ref[...]
    out2d_bf = out.reshape(nT * nB, D2).astype(jnp.bfloat16)

    aw = (jnp.dot(out2d_bf, waT_ref[...],
                  preferred_element_type=jnp.float32) + ba_ref[...])
    aw = jnp.exp(aw - jnp.max(aw, axis=-1, keepdims=True))
    aw = aw / jnp.sum(aw, axis=-1, keepdims=True)
    aw3 = aw.reshape(nT, nB, D2)

    ctx = jnp.sum(aw3 * out, axis=0)
    ctxp = jnp.dot(ctx.astype(jnp.bfloat16), wc1T_ref[...],
                   preferred_element_type=jnp.float32)
    comb = (jnp.dot(out2d_bf, wc2T_ref[...],
                    preferred_element_type=jnp.float32).reshape(nT, nB, D2)
            + ctxp[None] + bc_ref[...][None])
    logits2d = (jnp.dot(comb.reshape(nT * nB, D2).astype(jnp.bfloat16),
                        wlT_ref[...], preferred_element_type=jnp.float32)
                + bl_ref[...])
    logits_ref[...] = jnp.transpose(logits2d.reshape(nT, nB, nO), (1, 0, 2))

    am = jnp.mean(aw3, axis=-1)
    tids = lax.broadcasted_iota(jnp.int32, am.shape, 0)
    rel = jnp.zeros_like(am)
    cur = am
    for _ in range(min(3, nT)):
        mx = jnp.max(cur, axis=0, keepdims=True)
        cand = jnp.where(cur >= mx, tids, nT)
        first = jnp.min(cand, axis=0, keepdims=True)
        pick = (tids == first).astype(jnp.float32)
        rel = rel + pick
        cur = jnp.where(pick > 0, jnp.float32(-1e30), cur)

    h_last = hlast_ref[...]
    out_last = logits2d[(nT - 1) * nB:]
    hp = (jnp.dot(h_last.astype(jnp.bfloat16), whpT_ref[...],
                  preferred_element_type=jnp.float32) + bhp_ref[...])
    op = (jnp.dot(out_last.astype(jnp.bfloat16), wopT_ref[...],
                  preferred_element_type=jnp.float32) + bop_ref[...])

    probs = jnp.exp(op - jnp.max(op, axis=-1, keepdims=True))
    probs = probs / jnp.sum(probs, axis=-1, keepdims=True)
    ent = -jnp.sum(probs * jnp.log(probs + eps), axis=1, keepdims=True)

    dotp = jnp.sum(op * hp, axis=1, keepdims=True)
    n1 = jnp.sqrt(jnp.sum(op * op, axis=1, keepdims=True))
    n2 = jnp.sqrt(jnp.sum(hp * hp, axis=1, keepdims=True))
    sim = dotp / (jnp.maximum(n1, 1e-8) * jnp.maximum(n2, 1e-8))

    ap = jnp.exp(logits2d - jnp.max(logits2d, axis=-1, keepdims=True))
    ap = ap / jnp.sum(ap, axis=-1, keepdims=True)
    t_ent = -jnp.sum(ap * jnp.log(ap + eps), axis=-1,
                     keepdims=True).reshape(nT, nB)
    avg = jnp.mean(ap, axis=-1, keepdims=True).reshape(nT, nB)
    rd_terms = rel * avg * jnp.log(avg + eps)

    ones_t = jnp.ones((nT, 1), jnp.float32)
    dn_t = (((0,), (0,)), ((), ()))
    disp_t = lax.dot_general(t_ent, ones_t, dn_t,
                             precision=lax.Precision.HIGHEST) / nT
    rel_disp = -lax.dot_general(rd_terms, ones_t, dn_t,
                                precision=lax.Precision.HIGHEST)
    disp = (disp_t + rel_disp) * 0.5

    a1 = (jnp.dot(op.astype(jnp.bfloat16), wa1aT_ref[...],
                  preferred_element_type=jnp.float32)
          + jnp.dot(hp.astype(jnp.bfloat16), wa1bT_ref[...],
                    preferred_element_type=jnp.float32)
          + ba1_ref[...])
    a1 = jnp.maximum(a1, 0.0)
    score_pre = (jnp.dot(a1.astype(jnp.bfloat16), wa2T_ref[...],
                         preferred_element_type=jnp.float32) + ba2_ref[...])

    ent_ref[...] = ent
    sim_ref[...] = sim
    disp_ref[...] = disp

    combo = -0.05 * ent + 0.1 * sim + 0.1 * disp
    ones_col = jnp.ones_like(score_pre)
    dn = (((1,), (1,)), ((), ()))
    s = (lax.dot_general(score_pre, ones_col, dn,
                         preferred_element_type=jnp.float32)
         + lax.dot_general(ones_col, combo, dn,
                           preferred_element_type=jnp.float32))
    s = jnp.log(jnp.abs(s) + 1e-9) * jnp.sign(s)
    score_ref[...] = 2.0 * jax.nn.sigmoid(s) - 1.0


def kernel(embedding, lstm0_wihT, lstm0_whhT, lstm0_b,
           lstm1_wihT, lstm1_whhT, lstm1_b,
           waT, ba, wc1T, wc2T, bc, wlT, bl,
           whpT, bhp, wopT, bop,
           wa1aT, wa1bT, ba1, wa2T, ba2, tokens):
    nB, nT = tokens.shape

    out0, h0, c0 = _l0_call(tokens, embedding,
                            lstm0_wihT, lstm0_whhT, lstm0_b, nT, nB)
    out1, hn4, cn4 = _l1_call(out0, lstm1_wihT, lstm1_whhT, lstm1_b, h0, c0)

    h_n = hn4.reshape(4, nB, HID)
    c_n = cn4.reshape(4, nB, HID)

    logits, score, ent, sim, disp = pl.pallas_call(
        _head_kernel,
        out_shape=(jax.ShapeDtypeStruct((nB, nT, OUT), jnp.float32),
                   jax.ShapeDtypeStruct((nB, nB), jnp.float32),
                   jax.ShapeDtypeStruct((nB, 1), jnp.float32),
                   jax.ShapeDtypeStruct((nB, 1), jnp.float32),
                   jax.ShapeDtypeStruct((nB, 1), jnp.float32)),
        grid=(1,),
        in_specs=[pl.BlockSpec((nT, nB, 2 * HID), lambda i: (0, 0, 0)),
                  pl.BlockSpec((None, None, nB, HID), lambda i: (1, 1, 0, 0)),
                  ] + [_ANYVMEM] * 16,
        out_specs=(_ANYVMEM,) * 5,
        compiler_params=pltpu.CompilerParams(vmem_limit_bytes=_VMEM_LIMIT),
    )(out1, hn4,
      waT, ba, wc1T, wc2T, bc, wlT, bl,
      whpT, bhp, wopT, bop, wa1aT, wa1bT, ba1, wa2T, ba2)

    return (logits, (h_n, c_n), score,
            ent[:, 0], sim[:, 0], disp[:, 0])


# speedup vs baseline: 1.1636x; 1.1636x over previous
import jax
import jax.numpy as jnp
from jax import lax
from jax.experimental import pallas as pl
from jax.experimental.pallas import tpu as pltpu

HID = 512
OUT = 64

_VMEM_LIMIT = 56 * 1024 * 1024
_ANYVMEM = pl.BlockSpec(memory_space=pltpu.MemorySpace.VMEM)


def _gates(g, c):
    i_g = jax.nn.sigmoid(g[:, 0 * HID:1 * HID])
    f_g = jax.nn.sigmoid(g[:, 1 * HID:2 * HID])
    g_g = jnp.tanh(g[:, 2 * HID:3 * HID])
    o_g = jax.nn.sigmoid(g[:, 3 * HID:4 * HID])
    c_new = f_g * c + i_g * g_g
    return o_g * jnp.tanh(c_new), c_new


def _gates0(g):
    c_new = jax.nn.sigmoid(g[:, 0 * HID:1 * HID]) * jnp.tanh(g[:, 2 * HID:3 * HID])
    return jax.nn.sigmoid(g[:, 3 * HID:4 * HID]) * jnp.tanh(c_new), c_new


def _bidir_layer(nT, xpf_ref, xpb_ref, whhf, whhb, out_ref):
    H = HID
    hf, cf = _gates0(xpf_ref[0])
    hb, cb = _gates0(xpb_ref[nT - 1])
    out_ref[0, :, :H] = hf
    out_ref[nT - 1, :, H:] = hb
    for t in range(1, nT):
        gf = xpf_ref[t] + jnp.dot(hf.astype(jnp.bfloat16), whhf,
                                  preferred_element_type=jnp.float32)
        gb = xpb_ref[nT - 1 - t] + jnp.dot(hb.astype(jnp.bfloat16), whhb,
                                           preferred_element_type=jnp.float32)
        hf, cf = _gates(gf, cf)
        hb, cb = _gates(gb, cb)
        out_ref[t, :, :H] = hf
        out_ref[nT - 1 - t, :, H:] = hb
    return (hf, cf), (hb, cb)


def _net_kernel(tokens_ref, emb_ref,
                w0ih_ref, w0hh_ref, b0_ref,
                w1ih_ref, w1hh_ref, b1_ref,
                waT_ref, ba_ref, wc1T_ref, wc2T_ref, bc_ref, wlT_ref, bl_ref,
                whpT_ref, bhp_ref, wopT_ref, bop_ref,
                wa1aT_ref, wa1bT_ref, ba1_ref, wa2T_ref, ba2_ref,
                logits_ref, score_ref, ent_ref, sim_ref, disp_ref,
                hn_ref, cn_ref,
                xpf_ref, xpb_ref, out0_ref, out1_ref):
    eps = 1e-9
    nB, nT = tokens_ref.shape
    V, E = emb_ref.shape
    H = HID
    D2 = 2 * H

    tok_f = tokens_ref[...].astype(jnp.float32)
    eye_b = (lax.broadcasted_iota(jnp.int32, (nB, nB), 0)
             == lax.broadcasted_iota(jnp.int32, (nB, nB), 1)).astype(jnp.float32)
    tok_t = lax.dot_general(tok_f, eye_b, (((0,), (0,)), ((), ())),
                            preferred_element_type=jnp.float32)
    vids = lax.broadcasted_iota(jnp.int32, (nT, nB, V), 2)
    oh = (vids == tok_t.astype(jnp.int32)[:, :, None]).astype(jnp.bfloat16)
    x2d = jnp.dot(oh.reshape(nT * nB, V), emb_ref[...].astype(jnp.bfloat16),
                  preferred_element_type=jnp.float32).astype(jnp.bfloat16)

    xpf_ref[...] = (jnp.dot(x2d, w0ih_ref[0],
                            preferred_element_type=jnp.float32)
                    + b0_ref[0]).reshape(nT, nB, 4 * H)
    xpb_ref[...] = (jnp.dot(x2d, w0ih_ref[1],
                            preferred_element_type=jnp.float32)
                    + b0_ref[1]).reshape(nT, nB, 4 * H)
    (h0f, c0f), (h0b, c0b) = _bidir_layer(nT, xpf_ref, xpb_ref,
                                          w0hh_ref[0], w0hh_ref[1], out0_ref)
    hn_ref[0, 0] = h0f
    hn_ref[0, 1] = h0b
    cn_ref[0, 0] = c0f
    cn_ref[0, 1] = c0b

    x2d1 = out0_ref[...].reshape(nT * nB, D2).astype(jnp.bfloat16)
    xpf_ref[...] = (jnp.dot(x2d1, w1ih_ref[0],
                            preferred_element_type=jnp.float32)
                    + b1_ref[0]).reshape(nT, nB, 4 * H)
    xpb_ref[...] = (jnp.dot(x2d1, w1ih_ref[1],
                            preferred_element_type=jnp.float32)
                    + b1_ref[1]).reshape(nT, nB, 4 * H)
    (h1f, c1f), (h1b, c1b) = _bidir_layer(nT, xpf_ref, xpb_ref,
                                          w1hh_ref[0], w1hh_ref[1], out1_ref)
    hn_ref[1, 0] = h1f
    hn_ref[1, 1] = h1b
    cn_ref[1, 0] = c1f
    cn_ref[1, 1] = c1b

    out = out1_ref[...]
    out2d_bf = out.reshape(nT * nB, D2).astype(jnp.bfloat16)

    aw = (jnp.dot(out2d_bf, waT_ref[...],
                  preferred_element_type=jnp.float32) + ba_ref[...])
    aw = jnp.exp(aw - jnp.max(aw, axis=-1, keepdims=True))
    aw = aw / jnp.sum(aw, axis=-1, keepdims=True)
    aw3 = aw.reshape(nT, nB, D2)

    ctx = jnp.sum(aw3 * out, axis=0)
    ctxp = jnp.dot(ctx.astype(jnp.bfloat16), wc1T_ref[...],
                   preferred_element_type=jnp.float32)
    comb = (jnp.dot(out2d_bf, wc2T_ref[...],
                    preferred_element_type=jnp.float32).reshape(nT, nB, D2)
            + ctxp[None] + bc_ref[...][None])
    logits2d = (jnp.dot(comb.reshape(nT * nB, D2).astype(jnp.bfloat16),
                        wlT_ref[...], preferred_element_type=jnp.float32)
                + bl_ref[...])
    nO = logits_ref.shape[-1]
    logits_ref[...] = jnp.transpose(logits2d.reshape(nT, nB, nO), (1, 0, 2))

    am = jnp.mean(aw3, axis=-1)
    tids = lax.broadcasted_iota(jnp.int32, am.shape, 0)
    rel = jnp.zeros_like(am)
    cur = am
    for _ in range(min(3, nT)):
        mx = jnp.max(cur, axis=0, keepdims=True)
        cand = jnp.where(cur >= mx, tids, nT)
        first = jnp.min(cand, axis=0, keepdims=True)
        pick = (tids == first).astype(jnp.float32)
        rel = rel + pick
        cur = jnp.where(pick > 0, jnp.float32(-1e30), cur)

    out_last = logits2d[(nT - 1) * nB:]
    hp = (jnp.dot(h1b.astype(jnp.bfloat16), whpT_ref[...],
                  preferred_element_type=jnp.float32) + bhp_ref[...])
    op = (jnp.dot(out_last.astype(jnp.bfloat16), wopT_ref[...],
                  preferred_element_type=jnp.float32) + bop_ref[...])

    probs = jnp.exp(op - jnp.max(op, axis=-1, keepdims=True))
    probs = probs / jnp.sum(probs, axis=-1, keepdims=True)
    ent = -jnp.sum(probs * jnp.log(probs + eps), axis=1, keepdims=True)

    dotp = jnp.sum(op * hp, axis=1, keepdims=True)
    n1 = jnp.sqrt(jnp.sum(op * op, axis=1, keepdims=True))
    n2 = jnp.sqrt(jnp.sum(hp * hp, axis=1, keepdims=True))
    sim = dotp / (jnp.maximum(n1, 1e-8) * jnp.maximum(n2, 1e-8))

    ap = jnp.exp(logits2d - jnp.max(logits2d, axis=-1, keepdims=True))
    ap = ap / jnp.sum(ap, axis=-1, keepdims=True)
    t_ent = -jnp.sum(ap * jnp.log(ap + eps), axis=-1,
                     keepdims=True).reshape(nT, nB)
    avg = jnp.mean(ap, axis=-1, keepdims=True).reshape(nT, nB)
    rd_terms = rel * avg * jnp.log(avg + eps)

    ones_t = jnp.ones((nT, 1), jnp.float32)
    dn_t = (((0,), (0,)), ((), ()))
    disp_t = lax.dot_general(t_ent, ones_t, dn_t,
                             precision=lax.Precision.HIGHEST) / nT
    rel_disp = -lax.dot_general(rd_terms, ones_t, dn_t,
                                precision=lax.Precision.HIGHEST)
    disp = (disp_t + rel_disp) * 0.5

    a1 = (jnp.dot(op.astype(jnp.bfloat16), wa1aT_ref[...],
                  preferred_element_type=jnp.float32)
          + jnp.dot(hp.astype(jnp.bfloat16), wa1bT_ref[...],
                    preferred_element_type=jnp.float32)
          + ba1_ref[...])
    a1 = jnp.maximum(a1, 0.0)
    score_pre = (jnp.dot(a1.astype(jnp.bfloat16), wa2T_ref[...],
                         preferred_element_type=jnp.float32) + ba2_ref[...])

    ent_ref[...] = ent
    sim_ref[...] = sim
    disp_ref[...] = disp

    combo = -0.05 * ent + 0.1 * sim + 0.1 * disp
    ones_col = jnp.ones_like(score_pre)
    dn = (((1,), (1,)), ((), ()))
    s = (lax.dot_general(score_pre, ones_col, dn,
                         preferred_element_type=jnp.float32)
         + lax.dot_general(ones_col, combo, dn,
                           preferred_element_type=jnp.float32))
    s = jnp.log(jnp.abs(s) + 1e-9) * jnp.sign(s)
    score_ref[...] = 2.0 * jax.nn.sigmoid(s) - 1.0


def kernel(embedding, lstm0_wihT, lstm0_whhT, lstm0_b,
           lstm1_wihT, lstm1_whhT, lstm1_b,
           waT, ba, wc1T, wc2T, bc, wlT, bl,
           whpT, bhp, wopT, bop,
           wa1aT, wa1bT, ba1, wa2T, ba2, tokens):
    nB, nT = tokens.shape

    outs = pl.pallas_call(
        _net_kernel,
        out_shape=(jax.ShapeDtypeStruct((nB, nT, OUT), jnp.float32),
                   jax.ShapeDtypeStruct((nB, nB), jnp.float32),
                   jax.ShapeDtypeStruct((nB, 1), jnp.float32),
                   jax.ShapeDtypeStruct((nB, 1), jnp.float32),
                   jax.ShapeDtypeStruct((nB, 1), jnp.float32),
                   jax.ShapeDtypeStruct((2, 2, nB, HID), jnp.float32),
                   jax.ShapeDtypeStruct((2, 2, nB, HID), jnp.float32)),
        in_specs=[_ANYVMEM] * 24,
        out_specs=(_ANYVMEM,) * 7,
        scratch_shapes=[pltpu.VMEM((nT, nB, 4 * HID), jnp.float32),
                        pltpu.VMEM((nT, nB, 4 * HID), jnp.float32),
                        pltpu.VMEM((nT, nB, 2 * HID), jnp.float32),
                        pltpu.VMEM((nT, nB, 2 * HID), jnp.float32)],
        compiler_params=pltpu.CompilerParams(vmem_limit_bytes=_VMEM_LIMIT),
    )(tokens, embedding,
      lstm0_wihT, lstm0_whhT, lstm0_b,
      lstm1_wihT, lstm1_whhT, lstm1_b,
      waT, ba, wc1T, wc2T, bc, wlT, bl,
      whpT, bhp, wopT, bop,
      wa1aT, wa1bT, ba1, wa2T, ba2)

    logits, score, ent, sim, disp, hn4, cn4 = outs
    return (logits, (hn4.reshape(4, nB, HID), cn4.reshape(4, nB, HID)), score,
            ent[:, 0], sim[:, 0], disp[:, 0])


# speedup vs baseline: 1.3586x; 1.1676x over previous
import jax
import jax.numpy as jnp
from jax import lax
from jax.experimental import pallas as pl
from jax.experimental.pallas import tpu as pltpu

HID = 512
OUT = 64

_VMEM_LIMIT = 56 * 1024 * 1024
_ANYVMEM = pl.BlockSpec(memory_space=pltpu.MemorySpace.VMEM)


def _gates(g, c):
    i_g = jax.nn.sigmoid(g[:, 0 * HID:1 * HID])
    f_g = jax.nn.sigmoid(g[:, 1 * HID:2 * HID])
    g_g = jnp.tanh(g[:, 2 * HID:3 * HID])
    o_g = jax.nn.sigmoid(g[:, 3 * HID:4 * HID])
    c_new = f_g * c + i_g * g_g
    return o_g * jnp.tanh(c_new), c_new


def _gates0(g):
    c_new = jax.nn.sigmoid(g[:, 0 * HID:1 * HID]) * jnp.tanh(g[:, 2 * HID:3 * HID])
    return jax.nn.sigmoid(g[:, 3 * HID:4 * HID]) * jnp.tanh(c_new), c_new


def _bidir_layer(nT, xpf_ref, xpb_ref, whhf, whhb, out_ref):
    H = HID
    hf, cf = _gates0(xpf_ref[0])
    hb, cb = _gates0(xpb_ref[nT - 1])
    out_ref[0, :, :H] = hf
    out_ref[nT - 1, :, H:] = hb
    for t in range(1, nT):
        gf = xpf_ref[t] + jnp.dot(hf.astype(jnp.bfloat16), whhf,
                                  preferred_element_type=jnp.float32)
        gb = xpb_ref[nT - 1 - t] + jnp.dot(hb.astype(jnp.bfloat16), whhb,
                                           preferred_element_type=jnp.float32)
        hf, cf = _gates(gf, cf)
        hb, cb = _gates(gb, cb)
        out_ref[t, :, :H] = hf
        out_ref[nT - 1 - t, :, H:] = hb
    return (hf, cf), (hb, cb)


def _net_kernel(tokens_ref, emb_ref,
                w0ih_ref, w0hh_ref, b0_ref,
                w1ih_ref, w1hh_ref, b1_ref,
                waT_ref, ba_ref, wc1T_ref, wc2T_ref, bc_ref, wlT_ref, bl_ref,
                whpT_ref, bhp_ref, wopT_ref, bop_ref,
                wa1aT_ref, wa1bT_ref, ba1_ref, wa2T_ref, ba2_ref,
                logits_ref, score_ref, ent_ref, sim_ref, disp_ref,
                hn_ref, cn_ref,
                xpf_ref, xpb_ref, out0_ref, out1_ref,
                w0hh_s, w1ih_s, w1hh_s,
                wa_s, wc1_s, wc2_s, wl_s, whp_s, wop_s, wa1a_s, wa1b_s,
                sems):
    eps = 1e-9
    nB, nT = tokens_ref.shape
    V, E = emb_ref.shape
    H = HID
    D2 = 2 * H

    cp0 = pltpu.make_async_copy(w0hh_ref, w0hh_s, sems.at[0])
    cp1 = pltpu.make_async_copy(w1ih_ref, w1ih_s, sems.at[1])
    cp2 = pltpu.make_async_copy(w1hh_ref, w1hh_s, sems.at[2])
    cp0.start()
    cp1.start()
    cp2.start()
    head_srcs = (waT_ref, wc1T_ref, wc2T_ref, wlT_ref,
                 whpT_ref, wopT_ref, wa1aT_ref, wa1bT_ref)
    wh_s = (wa_s, wc1_s, wc2_s, wl_s, whp_s, wop_s, wa1a_s, wa1b_s)
    head_cps = tuple(
        pltpu.make_async_copy(src, dst, sems.at[3 + i])
        for i, (src, dst) in enumerate(zip(head_srcs, wh_s)))
    for cp in head_cps:
        cp.start()

    tok_f = tokens_ref[...].astype(jnp.float32)
    eye_b = (lax.broadcasted_iota(jnp.int32, (nB, nB), 0)
             == lax.broadcasted_iota(jnp.int32, (nB, nB), 1)).astype(jnp.float32)
    tok_t = lax.dot_general(tok_f, eye_b, (((0,), (0,)), ((), ())),
                            preferred_element_type=jnp.float32)
    vids = lax.broadcasted_iota(jnp.int32, (nT, nB, V), 2)
    oh = (vids == tok_t.astype(jnp.int32)[:, :, None]).astype(jnp.bfloat16)
    x2d = jnp.dot(oh.reshape(nT * nB, V), emb_ref[...].astype(jnp.bfloat16),
                  preferred_element_type=jnp.float32).astype(jnp.bfloat16)

    xpf_ref[...] = (jnp.dot(x2d, w0ih_ref[0],
                            preferred_element_type=jnp.float32)
                    + b0_ref[0]).reshape(nT, nB, 4 * H)
    xpb_ref[...] = (jnp.dot(x2d, w0ih_ref[1],
                            preferred_element_type=jnp.float32)
                    + b0_ref[1]).reshape(nT, nB, 4 * H)
    cp0.wait()
    (h0f, c0f), (h0b, c0b) = _bidir_layer(nT, xpf_ref, xpb_ref,
                                          w0hh_s[0], w0hh_s[1], out0_ref)
    hn_ref[0, 0] = h0f
    hn_ref[0, 1] = h0b
    cn_ref[0, 0] = c0f
    cn_ref[0, 1] = c0b

    cp1.wait()
    x2d1 = out0_ref[...].reshape(nT * nB, D2).astype(jnp.bfloat16)
    xpf_ref[...] = (jnp.dot(x2d1, w1ih_s[0],
                            preferred_element_type=jnp.float32)
                    + b1_ref[0]).reshape(nT, nB, 4 * H)
    xpb_ref[...] = (jnp.dot(x2d1, w1ih_s[1],
                            preferred_element_type=jnp.float32)
                    + b1_ref[1]).reshape(nT, nB, 4 * H)
    cp2.wait()
    (h1f, c1f), (h1b, c1b) = _bidir_layer(nT, xpf_ref, xpb_ref,
                                          w1hh_s[0], w1hh_s[1], out1_ref)
    hn_ref[1, 0] = h1f
    hn_ref[1, 1] = h1b
    cn_ref[1, 0] = c1f
    cn_ref[1, 1] = c1b

    for cp in head_cps:
        cp.wait()
    out = out1_ref[...]
    out2d_bf = out.reshape(nT * nB, D2).astype(jnp.bfloat16)

    aw = (jnp.dot(out2d_bf, wa_s[...],
                  preferred_element_type=jnp.float32) + ba_ref[...])
    aw = jnp.exp(aw - jnp.max(aw, axis=-1, keepdims=True))
    aw = aw / jnp.sum(aw, axis=-1, keepdims=True)
    aw3 = aw.reshape(nT, nB, D2)

    ctx = jnp.sum(aw3 * out, axis=0)
    ctxp = jnp.dot(ctx.astype(jnp.bfloat16), wc1_s[...],
                   preferred_element_type=jnp.float32)
    comb = (jnp.dot(out2d_bf, wc2_s[...],
                    preferred_element_type=jnp.float32).reshape(nT, nB, D2)
            + ctxp[None] + bc_ref[...][None])
    logits2d = (jnp.dot(comb.reshape(nT * nB, D2).astype(jnp.bfloat16),
                        wl_s[...], preferred_element_type=jnp.float32)
                + bl_ref[...])
    nO = logits_ref.shape[-1]
    logits_ref[...] = jnp.transpose(logits2d.reshape(nT, nB, nO), (1, 0, 2))

    am = jnp.mean(aw3, axis=-1)
    tids = lax.broadcasted_iota(jnp.int32, am.shape, 0)
    rel = jnp.zeros_like(am)
    cur = am
    for _ in range(min(3, nT)):
        mx = jnp.max(cur, axis=0, keepdims=True)
        cand = jnp.where(cur >= mx, tids, nT)
        first = jnp.min(cand, axis=0, keepdims=True)
        pick = (tids == first).astype(jnp.float32)
        rel = rel + pick
        cur = jnp.where(pick > 0, jnp.float32(-1e30), cur)

    out_last = logits2d[(nT - 1) * nB:]
    hp = (jnp.dot(h1b.astype(jnp.bfloat16), whp_s[...],
                  preferred_element_type=jnp.float32) + bhp_ref[...])
    op = (jnp.dot(out_last.astype(jnp.bfloat16), wop_s[...],
                  preferred_element_type=jnp.float32) + bop_ref[...])

    probs = jnp.exp(op - jnp.max(op, axis=-1, keepdims=True))
    probs = probs / jnp.sum(probs, axis=-1, keepdims=True)
    ent = -jnp.sum(probs * jnp.log(probs + eps), axis=1, keepdims=True)

    dotp = jnp.sum(op * hp, axis=1, keepdims=True)
    n1 = jnp.sqrt(jnp.sum(op * op, axis=1, keepdims=True))
    n2 = jnp.sqrt(jnp.sum(hp * hp, axis=1, keepdims=True))
    sim = dotp / (jnp.maximum(n1, 1e-8) * jnp.maximum(n2, 1e-8))

    ap = jnp.exp(logits2d - jnp.max(logits2d, axis=-1, keepdims=True))
    ap = ap / jnp.sum(ap, axis=-1, keepdims=True)
    t_ent = -jnp.sum(ap * jnp.log(ap + eps), axis=-1,
                     keepdims=True).reshape(nT, nB)
    avg = jnp.mean(ap, axis=-1, keepdims=True).reshape(nT, nB)
    rd_terms = rel * avg * jnp.log(avg + eps)

    ones_t = jnp.ones((nT, 1), jnp.float32)
    dn_t = (((0,), (0,)), ((), ()))
    disp_t = lax.dot_general(t_ent, ones_t, dn_t,
                             precision=lax.Precision.HIGHEST) / nT
    rel_disp = -lax.dot_general(rd_terms, ones_t, dn_t,
                                precision=lax.Precision.HIGHEST)
    disp = (disp_t + rel_disp) * 0.5

    a1 = (jnp.dot(op.astype(jnp.bfloat16), wa1a_s[...],
                  preferred_element_type=jnp.float32)
          + jnp.dot(hp.astype(jnp.bfloat16), wa1b_s[...],
                    preferred_element_type=jnp.float32)
          + ba1_ref[...])
    a1 = jnp.maximum(a1, 0.0)
    score_pre = (jnp.dot(a1.astype(jnp.bfloat16), wa2T_ref[...],
                         preferred_element_type=jnp.float32) + ba2_ref[...])

    one11 = jnp.ones((1, 1), jnp.float32)
    dn_c1 = (((1,), (1,)), ((), ()))
    ent_ref[...] = lax.dot_general(one11, ent, dn_c1,
                                   precision=lax.Precision.HIGHEST)
    sim_ref[...] = lax.dot_general(one11, sim, dn_c1,
                                   precision=lax.Precision.HIGHEST)
    disp_ref[...] = lax.dot_general(one11, disp, dn_c1,
                                    precision=lax.Precision.HIGHEST)

    combo = -0.05 * ent + 0.1 * sim + 0.1 * disp
    ones_col = jnp.ones_like(score_pre)
    dn = (((1,), (1,)), ((), ()))
    s = (lax.dot_general(score_pre, ones_col, dn,
                         preferred_element_type=jnp.float32)
         + lax.dot_general(ones_col, combo, dn,
                           preferred_element_type=jnp.float32))
    s = jnp.log(jnp.abs(s) + 1e-9) * jnp.sign(s)
    score_ref[...] = 2.0 * jax.nn.sigmoid(s) - 1.0


def kernel(embedding, lstm0_wihT, lstm0_whhT, lstm0_b,
           lstm1_wihT, lstm1_whhT, lstm1_b,
           waT, ba, wc1T, wc2T, bc, wlT, bl,
           whpT, bhp, wopT, bop,
           wa1aT, wa1bT, ba1, wa2T, ba2, tokens):
    nB, nT = tokens.shape

    outs = pl.pallas_call(
        _net_kernel,
        out_shape=(jax.ShapeDtypeStruct((nB, nT, OUT), jnp.float32),
                   jax.ShapeDtypeStruct((nB, nB), jnp.float32),
                   jax.ShapeDtypeStruct((1, nB), jnp.float32),
                   jax.ShapeDtypeStruct((1, nB), jnp.float32),
                   jax.ShapeDtypeStruct((1, nB), jnp.float32),
                   jax.ShapeDtypeStruct((2, 2, nB, HID), jnp.float32),
                   jax.ShapeDtypeStruct((2, 2, nB, HID), jnp.float32)),
        in_specs=([_ANYVMEM] * 3
                  + [pl.BlockSpec(memory_space=pltpu.MemorySpace.HBM)]
                  + [_ANYVMEM]
                  + [pl.BlockSpec(memory_space=pltpu.MemorySpace.HBM)] * 2
                  + [_ANYVMEM]
                  + [pl.BlockSpec(memory_space=pltpu.MemorySpace.HBM),
                     _ANYVMEM,
                     pl.BlockSpec(memory_space=pltpu.MemorySpace.HBM),
                     pl.BlockSpec(memory_space=pltpu.MemorySpace.HBM),
                     _ANYVMEM,
                     pl.BlockSpec(memory_space=pltpu.MemorySpace.HBM),
                     _ANYVMEM,
                     pl.BlockSpec(memory_space=pltpu.MemorySpace.HBM),
                     _ANYVMEM,
                     pl.BlockSpec(memory_space=pltpu.MemorySpace.HBM),
                     _ANYVMEM,
                     pl.BlockSpec(memory_space=pltpu.MemorySpace.HBM),
                     pl.BlockSpec(memory_space=pltpu.MemorySpace.HBM),
                     _ANYVMEM, _ANYVMEM, _ANYVMEM]),
        out_specs=(_ANYVMEM,) * 7,
        scratch_shapes=[pltpu.VMEM((nT, nB, 4 * HID), jnp.float32),
                        pltpu.VMEM((nT, nB, 4 * HID), jnp.float32),
                        pltpu.VMEM((nT, nB, 2 * HID), jnp.float32),
                        pltpu.VMEM((nT, nB, 2 * HID), jnp.float32),
                        pltpu.VMEM((2, HID, 4 * HID), jnp.bfloat16),
                        pltpu.VMEM((2, 2 * HID, 4 * HID), jnp.bfloat16),
                        pltpu.VMEM((2, HID, 4 * HID), jnp.bfloat16),
                        pltpu.VMEM((2 * HID, 2 * HID), jnp.bfloat16),
                        pltpu.VMEM((2 * HID, 2 * HID), jnp.bfloat16),
                        pltpu.VMEM((2 * HID, 2 * HID), jnp.bfloat16),
                        pltpu.VMEM((2 * HID, OUT), jnp.bfloat16),
                        pltpu.VMEM((HID, 768), jnp.bfloat16),
                        pltpu.VMEM((OUT, 768), jnp.bfloat16),
                        pltpu.VMEM((768, HID), jnp.bfloat16),
                        pltpu.VMEM((768, HID), jnp.bfloat16),
                        pltpu.SemaphoreType.DMA((11,))],
        compiler_params=pltpu.CompilerParams(vmem_limit_bytes=_VMEM_LIMIT),
    )(tokens, embedding,
      lstm0_wihT, lstm0_whhT, lstm0_b,
      lstm1_wihT, lstm1_whhT, lstm1_b,
      waT, ba, wc1T, wc2T, bc, wlT, bl,
      whpT, bhp, wopT, bop,
      wa1aT, wa1bT, ba1, wa2T, ba2)

    logits, score, ent, sim, disp, hn4, cn4 = outs
    return (logits, (hn4.reshape(4, nB, HID), cn4.reshape(4, nB, HID)), score,
            ent.reshape(nB), sim.reshape(nB), disp.reshape(nB))


# speedup vs baseline: 1.6012x; 1.1785x over previous
import jax
import jax.numpy as jnp
from jax import lax
from jax.experimental import pallas as pl
from jax.experimental.pallas import tpu as pltpu

HID = 512
OUT = 64

_VMEM_LIMIT = 56 * 1024 * 1024
_ANYVMEM = pl.BlockSpec(memory_space=pltpu.MemorySpace.VMEM)


def _gates(g, c):
    i_g = jax.nn.sigmoid(g[:, 0 * HID:1 * HID])
    f_g = jax.nn.sigmoid(g[:, 1 * HID:2 * HID])
    g_g = jnp.tanh(g[:, 2 * HID:3 * HID])
    o_g = jax.nn.sigmoid(g[:, 3 * HID:4 * HID])
    c_new = f_g * c + i_g * g_g
    return o_g * jnp.tanh(c_new), c_new


def _gates0(g):
    c_new = jax.nn.sigmoid(g[:, 0 * HID:1 * HID]) * jnp.tanh(g[:, 2 * HID:3 * HID])
    return jax.nn.sigmoid(g[:, 3 * HID:4 * HID]) * jnp.tanh(c_new), c_new


def _bidir_layer(nT, xpf_ref, xpb_ref, whhf, whhb, out_ref):
    H = HID
    hf, cf = _gates0(xpf_ref[0])
    hb, cb = _gates0(xpb_ref[nT - 1])
    out_ref[0, :, :H] = hf
    out_ref[nT - 1, :, H:] = hb
    for t in range(1, nT):
        gf = xpf_ref[t] + jnp.dot(hf.astype(jnp.bfloat16), whhf,
                                  preferred_element_type=jnp.float32)
        gb = xpb_ref[nT - 1 - t] + jnp.dot(hb.astype(jnp.bfloat16), whhb,
                                           preferred_element_type=jnp.float32)
        hf, cf = _gates(gf, cf)
        hb, cb = _gates(gb, cb)
        out_ref[t, :, :H] = hf
        out_ref[nT - 1 - t, :, H:] = hb
    return (hf, cf), (hb, cb)


def _net_kernel(tokens_ref, emb_ref,
                w0ih_ref, w0hh_ref, b0_ref,
                w1ih_ref, w1hh_ref, b1_ref,
                waT_ref, ba_ref, wc1T_ref, wc2T_ref, bc_ref, wlT_ref, bl_ref,
                whpT_ref, bhp_ref, wopT_ref, bop_ref,
                wa1aT_ref, wa1bT_ref, ba1_ref, wa2T_ref, ba2_ref,
                logits_ref, score_ref, ent_ref, sim_ref, disp_ref,
                hn_ref, cn_ref,
                xpf_ref, xpb_ref, out0_ref, out1_ref,
                w0hh_s, w1ih_s, w1hh_s,
                wa_s, wc1_s, wc2_s, wl_s, whp_s, wop_s, wa1a_s, wa1b_s,
                sems):
    eps = 1e-9
    nB, nT = tokens_ref.shape
    E, V = emb_ref.shape
    H = HID
    D2 = 2 * H

    cp0 = pltpu.make_async_copy(w0hh_ref, w0hh_s, sems.at[0])
    cp1 = pltpu.make_async_copy(w1ih_ref, w1ih_s, sems.at[1])
    cp2 = pltpu.make_async_copy(w1hh_ref, w1hh_s, sems.at[2])
    cp0.start()
    cp1.start()
    cp2.start()
    head_srcs = (waT_ref, wc1T_ref, wc2T_ref, wlT_ref,
                 whpT_ref, wopT_ref, wa1aT_ref, wa1bT_ref)
    wh_s = (wa_s, wc1_s, wc2_s, wl_s, whp_s, wop_s, wa1a_s, wa1b_s)
    head_cps = tuple(
        pltpu.make_async_copy(src, dst, sems.at[3 + i])
        for i, (src, dst) in enumerate(zip(head_srcs, wh_s)))

    tok_f = tokens_ref[...].astype(jnp.float32)
    eye_b = (lax.broadcasted_iota(jnp.int32, (nB, nB), 0)
             == lax.broadcasted_iota(jnp.int32, (nB, nB), 1)).astype(jnp.float32)
    tok_t = lax.dot_general(tok_f, eye_b, (((0,), (0,)), ((), ())),
                            preferred_element_type=jnp.float32)
    vids = lax.broadcasted_iota(jnp.int32, (nT, nB, V), 2)
    oh = (vids == tok_t.astype(jnp.int32)[:, :, None]).astype(jnp.bfloat16)
    x2d = lax.dot_general(oh.reshape(nT * nB, V),
                          emb_ref[...].astype(jnp.bfloat16),
                          (((1,), (1,)), ((), ())),
                          preferred_element_type=jnp.float32).astype(jnp.bfloat16)

    xpf_ref[...] = (jnp.dot(x2d, w0ih_ref[0],
                            preferred_element_type=jnp.float32)
                    + b0_ref[0]).reshape(nT, nB, 4 * H)
    xpb_ref[...] = (jnp.dot(x2d, w0ih_ref[1],
                            preferred_element_type=jnp.float32)
                    + b0_ref[1]).reshape(nT, nB, 4 * H)
    cp0.wait()
    (h0f, c0f), (h0b, c0b) = _bidir_layer(nT, xpf_ref, xpb_ref,
                                          w0hh_s[0], w0hh_s[1], out0_ref)
    hn_ref[0, 0] = h0f
    hn_ref[0, 1] = h0b
    cn_ref[0, 0] = c0f
    cn_ref[0, 1] = c0b

    cp1.wait()
    for cp in head_cps:
        cp.start()
    cp2.wait()
    x2d1 = out0_ref[...].reshape(nT * nB, D2).astype(jnp.bfloat16)
    CH = 8
    nC = nT // CH

    def _xp_chunk(dst_ref, w, b, c):
        rows = x2d1[c * CH * nB:(c + 1) * CH * nB]
        dst_ref[c * CH:(c + 1) * CH] = (
            jnp.dot(rows, w, preferred_element_type=jnp.float32) + b
        ).reshape(CH, nB, 4 * H)

    _xp_chunk(xpf_ref, w1ih_s[0], b1_ref[0], 0)
    _xp_chunk(xpb_ref, w1ih_s[1], b1_ref[1], nC - 1)
    whhf, whhb = w1hh_s[0], w1hh_s[1]
    h1f, c1f = _gates0(xpf_ref[0])
    h1b, c1b = _gates0(xpb_ref[nT - 1])
    out1_ref[0, :, :H] = h1f
    out1_ref[nT - 1, :, H:] = h1b
    done_f = done_b = 1
    for t in range(1, nT):
        if done_f < nC and t == (done_f - 1) * CH + 1:
            _xp_chunk(xpf_ref, w1ih_s[0], b1_ref[0], done_f)
            done_f += 1
        if done_b < nC and t == (done_b - 1) * CH + 1:
            _xp_chunk(xpb_ref, w1ih_s[1], b1_ref[1], nC - 1 - done_b)
            done_b += 1
        gf = xpf_ref[t] + jnp.dot(h1f.astype(jnp.bfloat16), whhf,
                                  preferred_element_type=jnp.float32)
        gb = xpb_ref[nT - 1 - t] + jnp.dot(h1b.astype(jnp.bfloat16), whhb,
                                           preferred_element_type=jnp.float32)
        h1f, c1f = _gates(gf, c1f)
        h1b, c1b = _gates(gb, c1b)
        out1_ref[t, :, :H] = h1f
        out1_ref[nT - 1 - t, :, H:] = h1b
    hn_ref[1, 0] = h1f
    hn_ref[1, 1] = h1b
    cn_ref[1, 0] = c1f
    cn_ref[1, 1] = c1b

    for cp in head_cps:
        cp.wait()
    out = out1_ref[...]
    out2d_bf = out.reshape(nT * nB, D2).astype(jnp.bfloat16)

    aw = (jnp.dot(out2d_bf, wa_s[...],
                  preferred_element_type=jnp.float32) + ba_ref[...])
    aw = jnp.exp(aw - jnp.max(aw, axis=-1, keepdims=True))
    aw = aw / jnp.sum(aw, axis=-1, keepdims=True)
    aw3 = aw.reshape(nT, nB, D2)

    ctx = jnp.sum(aw3 * out, axis=0)
    ctxp = jnp.dot(ctx.astype(jnp.bfloat16), wc1_s[...],
                   preferred_element_type=jnp.float32)
    comb = (jnp.dot(out2d_bf, wc2_s[...],
                    preferred_element_type=jnp.float32).reshape(nT, nB, D2)
            + ctxp[None] + bc_ref[...][None])
    logits2d = (lax.dot_general(
                    comb.reshape(nT * nB, D2).astype(jnp.bfloat16),
                    wl_s[...], (((1,), (1,)), ((), ())),
                    preferred_element_type=jnp.float32)
                + bl_ref[...])
    nO = logits_ref.shape[-1]
    logits_ref[...] = jnp.transpose(logits2d.reshape(nT, nB, nO), (1, 0, 2))

    am = jnp.mean(aw3, axis=-1)
    tids = lax.broadcasted_iota(jnp.int32, am.shape, 0)
    rel = jnp.zeros_like(am)
    cur = am
    for _ in range(min(3, nT)):
        mx = jnp.max(cur, axis=0, keepdims=True)
        cand = jnp.where(cur >= mx, tids, nT)
        first = jnp.min(cand, axis=0, keepdims=True)
        pick = (tids == first).astype(jnp.float32)
        rel = rel + pick
        cur = jnp.where(pick > 0, jnp.float32(-1e30), cur)

    out_last = logits2d[(nT - 1) * nB:]
    hp = (jnp.dot(h1b.astype(jnp.bfloat16), whp_s[...],
                  preferred_element_type=jnp.float32) + bhp_ref[...])
    op = (jnp.dot(out_last.astype(jnp.bfloat16), wop_s[...],
                  preferred_element_type=jnp.float32) + bop_ref[...])

    probs = jnp.exp(op - jnp.max(op, axis=-1, keepdims=True))
    probs = probs / jnp.sum(probs, axis=-1, keepdims=True)
    ent = -jnp.sum(probs * jnp.log(probs + eps), axis=1, keepdims=True)

    dotp = jnp.sum(op * hp, axis=1, keepdims=True)
    n1 = jnp.sqrt(jnp.sum(op * op, axis=1, keepdims=True))
    n2 = jnp.sqrt(jnp.sum(hp * hp, axis=1, keepdims=True))
    sim = dotp / (jnp.maximum(n1, 1e-8) * jnp.maximum(n2, 1e-8))

    ap = jnp.exp(logits2d - jnp.max(logits2d, axis=-1, keepdims=True))
    ap = ap / jnp.sum(ap, axis=-1, keepdims=True)
    t_ent = -jnp.sum(ap * jnp.log(ap + eps), axis=-1,
                     keepdims=True).reshape(nT, nB)
    avg = jnp.mean(ap, axis=-1, keepdims=True).reshape(nT, nB)
    rd_terms = rel * avg * jnp.log(avg + eps)

    ones_t = jnp.ones((nT, 1), jnp.float32)
    dn_t = (((0,), (0,)), ((), ()))
    disp_t = lax.dot_general(t_ent, ones_t, dn_t,
                             precision=lax.Precision.HIGHEST) / nT
    rel_disp = -lax.dot_general(rd_terms, ones_t, dn_t,
                                precision=lax.Precision.HIGHEST)
    disp = (disp_t + rel_disp) * 0.5

    a1 = (jnp.dot(op.astype(jnp.bfloat16), wa1a_s[...],
                  preferred_element_type=jnp.float32)
          + jnp.dot(hp.astype(jnp.bfloat16), wa1b_s[...],
                    preferred_element_type=jnp.float32)
          + ba1_ref[...])
    a1 = jnp.maximum(a1, 0.0)
    spr = (lax.dot_general(wa2T_ref[...], a1.astype(jnp.bfloat16),
                           (((1,), (1,)), ((), ())),
                           preferred_element_type=jnp.float32)
           + ba2_ref[...])

    one11 = jnp.ones((1, 1), jnp.float32)
    dn_c1 = (((1,), (1,)), ((), ()))
    ent_ref[...] = lax.dot_general(one11, ent, dn_c1,
                                   precision=lax.Precision.HIGHEST)
    sim_ref[...] = lax.dot_general(one11, sim, dn_c1,
                                   precision=lax.Precision.HIGHEST)
    disp_ref[...] = lax.dot_general(one11, disp, dn_c1,
                                    precision=lax.Precision.HIGHEST)

    combo = -0.05 * ent + 0.1 * sim + 0.1 * disp
    ones_row = jnp.ones_like(spr)
    ones_col = jnp.ones((spr.shape[1], 1), jnp.float32)
    s = (lax.dot_general(spr, ones_row, (((0,), (0,)), ((), ())),
                         preferred_element_type=jnp.float32)
         + lax.dot_general(ones_col, combo, (((1,), (1,)), ((), ())),
                           preferred_element_type=jnp.float32))
    s = jnp.log(jnp.abs(s) + 1e-9) * jnp.sign(s)
    score_ref[...] = 2.0 * jax.nn.sigmoid(s) - 1.0


def kernel(embedding, lstm0_wihT, lstm0_whhT, lstm0_b,
           lstm1_wihT, lstm1_whhT, lstm1_b,
           waT, ba, wc1T, wc2T, bc, wlT, bl,
           whpT, bhp, wopT, bop,
           wa1aT, wa1bT, ba1, wa2T, ba2, tokens):
    nB, nT = tokens.shape

    outs = pl.pallas_call(
        _net_kernel,
        out_shape=(jax.ShapeDtypeStruct((nB, nT, OUT), jnp.float32),
                   jax.ShapeDtypeStruct((nB, nB), jnp.float32),
                   jax.ShapeDtypeStruct((1, nB), jnp.float32),
                   jax.ShapeDtypeStruct((1, nB), jnp.float32),
                   jax.ShapeDtypeStruct((1, nB), jnp.float32),
                   jax.ShapeDtypeStruct((2, 2, nB, HID), jnp.float32),
                   jax.ShapeDtypeStruct((2, 2, nB, HID), jnp.float32)),
        in_specs=([_ANYVMEM] * 3
                  + [pl.BlockSpec(memory_space=pltpu.MemorySpace.HBM)]
                  + [_ANYVMEM]
                  + [pl.BlockSpec(memory_space=pltpu.MemorySpace.HBM)] * 2
                  + [_ANYVMEM]
                  + [pl.BlockSpec(memory_space=pltpu.MemorySpace.HBM),
                     _ANYVMEM,
                     pl.BlockSpec(memory_space=pltpu.MemorySpace.HBM),
                     pl.BlockSpec(memory_space=pltpu.MemorySpace.HBM),
                     _ANYVMEM,
                     pl.BlockSpec(memory_space=pltpu.MemorySpace.HBM),
                     _ANYVMEM,
                     pl.BlockSpec(memory_space=pltpu.MemorySpace.HBM),
                     _ANYVMEM,
                     pl.BlockSpec(memory_space=pltpu.MemorySpace.HBM),
                     _ANYVMEM,
                     pl.BlockSpec(memory_space=pltpu.MemorySpace.HBM),
                     pl.BlockSpec(memory_space=pltpu.MemorySpace.HBM),
                     _ANYVMEM, _ANYVMEM, _ANYVMEM]),
        out_specs=(_ANYVMEM,) * 7,
        scratch_shapes=[pltpu.VMEM((nT, nB, 4 * HID), jnp.float32),
                        pltpu.VMEM((nT, nB, 4 * HID), jnp.float32),
                        pltpu.VMEM((nT, nB, 2 * HID), jnp.float32),
                        pltpu.VMEM((nT, nB, 2 * HID), jnp.float32),
                        pltpu.VMEM((2, HID, 4 * HID), jnp.bfloat16),
                        pltpu.VMEM((2, 2 * HID, 4 * HID), jnp.bfloat16),
                        pltpu.VMEM((2, HID, 4 * HID), jnp.bfloat16),
                        pltpu.VMEM((2 * HID, 2 * HID), jnp.bfloat16),
                        pltpu.VMEM((2 * HID, 2 * HID), jnp.bfloat16),
                        pltpu.VMEM((2 * HID, 2 * HID), jnp.bfloat16),
                        pltpu.VMEM((OUT, 2 * HID), jnp.bfloat16),
                        pltpu.VMEM((HID, 768), jnp.bfloat16),
                        pltpu.VMEM((OUT, 768), jnp.bfloat16),
                        pltpu.VMEM((768, HID), jnp.bfloat16),
                        pltpu.VMEM((768, HID), jnp.bfloat16),
                        pltpu.SemaphoreType.DMA((11,))],
        compiler_params=pltpu.CompilerParams(vmem_limit_bytes=_VMEM_LIMIT),
    )(tokens, embedding.T,
      lstm0_wihT, lstm0_whhT, lstm0_b,
      lstm1_wihT, lstm1_whhT, lstm1_b,
      waT, ba, wc1T, wc2T, bc, wlT.T, bl,
      whpT, bhp, wopT, bop,
      wa1aT, wa1bT, ba1, wa2T.T, ba2)

    logits, score, ent, sim, disp, hn4, cn4 = outs
    return (logits, (hn4.reshape(4, nB, HID), cn4.reshape(4, nB, HID)), score,
            ent.reshape(nB), sim.reshape(nB), disp.reshape(nB))
